# baseline (speedup 1.0000x reference)
import functools
import math

import jax
import jax.numpy as jnp
from jax import lax
from jax.experimental import pallas as pl
from jax.experimental.pallas import tpu as pltpu

BLK = 64
TOPK_RATIO = 0.125


def _lut_kernel(q_ref, k_ref, lut_ref, *, nb, topk, blk, d):
    qh = q_ref[0]
    kh = k_ref[0]
    pooled_q = qh.reshape(nb, blk, d).mean(axis=1)
    kmean = kh.mean(axis=0, keepdims=True)
    pooled_k = kh.reshape(nb, blk, d).mean(axis=1) - kmean
    scores = lax.dot_general(
        pooled_q, pooled_k, (((1,), (1,)), ((), ())),
        preferred_element_type=jnp.float32)

    col_iota = lax.broadcasted_iota(jnp.int32, (nb, nb), 1)
    out_iota = lax.broadcasted_iota(jnp.int32, (nb, topk), 1)
    lut = jnp.zeros((nb, topk), jnp.int32)
    s = scores
    neg = jnp.float32(-jnp.inf)
    for t in range(topk):
        mx = jnp.max(s, axis=1, keepdims=True)
        cand = jnp.where(s == mx, col_iota, nb)
        idx = jnp.min(cand, axis=1)
        lut = jnp.where(out_iota == t, idx[:, None], lut)
        s = jnp.where(col_iota == idx[:, None], neg, s)
    lut_ref[0] = lut


def _attn_kernel(lut_ref, q_ref, k_ref, v_ref, o_ref, *, nb, topk, blk, d):
    scale = 1.0 / math.sqrt(d)
    npair = topk // 2

    def body(m, _):
        q_m = q_ref[0, pl.ds(m * blk, blk), :]
        s_parts = []
        v_parts = []
        for jp in range(npair):
            i0 = lut_ref[0, m, 2 * jp]
            i1 = lut_ref[0, m, 2 * jp + 1]
            kp = jnp.concatenate(
                [k_ref[0, pl.ds(i0 * blk, blk), :],
                 k_ref[0, pl.ds(i1 * blk, blk), :]], axis=0)
            vp = jnp.concatenate(
                [v_ref[0, pl.ds(i0 * blk, blk), :],
                 v_ref[0, pl.ds(i1 * blk, blk), :]], axis=0)
            s_parts.append(lax.dot_general(
                q_m, kp, (((1,), (1,)), ((), ())),
                preferred_element_type=jnp.float32))
            v_parts.append(vp)
        if topk % 2:
            i0 = lut_ref[0, m, topk - 1]
            s_parts.append(lax.dot_general(
                q_m, k_ref[0, pl.ds(i0 * blk, blk), :],
                (((1,), (1,)), ((), ())),
                preferred_element_type=jnp.float32))
            v_parts.append(v_ref[0, pl.ds(i0 * blk, blk), :])
        s_parts = [sp * scale for sp in s_parts]
        mx = s_parts[0].max(axis=1, keepdims=True)
        for sp in s_parts[1:]:
            mx = jnp.maximum(mx, sp.max(axis=1, keepdims=True))
        p_parts = [jnp.exp(sp - mx) for sp in s_parts]
        l = p_parts[0].sum(axis=1, keepdims=True)
        for pp in p_parts[1:]:
            l = l + pp.sum(axis=1, keepdims=True)
        linv = 1.0 / l
        o = jnp.zeros((blk, d), jnp.float32)
        for pp, vp in zip(p_parts, v_parts):
            o = o + lax.dot_general(
                pp * linv, vp, (((1,), (0,)), ((), ())),
                preferred_element_type=jnp.float32)
        o_ref[0, pl.ds(m * blk, blk), :] = o
        return 0

    lax.fori_loop(0, nb, body, 0)


def kernel(q, k, v):
    B, H, L, D = q.shape
    nb = L // BLK
    topk = max(1, int(nb * TOPK_RATIO))
    BH = B * H
    qf = q.reshape(BH, L, D)
    kf = k.reshape(BH, L, D)
    vf = v.reshape(BH, L, D)

    head_spec = pl.BlockSpec((1, L, D), lambda i: (i, 0, 0))

    lut = pl.pallas_call(
        functools.partial(_lut_kernel, nb=nb, topk=topk, blk=BLK, d=D),
        grid=(BH,),
        in_specs=[head_spec, head_spec],
        out_specs=pl.BlockSpec((1, nb, topk), lambda i: (i, 0, 0)),
        out_shape=jax.ShapeDtypeStruct((BH, nb, topk), jnp.int32),
        compiler_params=pltpu.CompilerParams(
            dimension_semantics=("parallel",)),
    )(qf, kf)

    o = pl.pallas_call(
        functools.partial(_attn_kernel, nb=nb, topk=topk, blk=BLK, d=D),
        grid=(BH,),
        in_specs=[
            pl.BlockSpec((1, nb, topk), lambda i: (i, 0, 0),
                         memory_space=pltpu.SMEM),
            head_spec, head_spec, head_spec,
        ],
        out_specs=head_spec,
        out_shape=jax.ShapeDtypeStruct((BH, L, D), jnp.float32),
        compiler_params=pltpu.CompilerParams(
            dimension_semantics=("parallel",)),
    )(lut, qf, kf, vf)

    return o.reshape(B, H, L, D)


# speedup vs baseline: 1.1773x; 1.1773x over previous
import functools
import math

import jax
import jax.numpy as jnp
from jax import lax
from jax.experimental import pallas as pl
from jax.experimental.pallas import tpu as pltpu

BLK = 64
TOPK_RATIO = 0.125


def _lut_kernel(q_ref, k_ref, lut_ref, *, nb, topk, blk, d):
    qh = q_ref[0]
    kh = k_ref[0]
    pooled_q = qh.reshape(nb, blk, d).mean(axis=1)
    kmean = kh.mean(axis=0, keepdims=True)
    pooled_k = kh.reshape(nb, blk, d).mean(axis=1) - kmean
    scores = lax.dot_general(
        pooled_q, pooled_k, (((1,), (1,)), ((), ())),
        preferred_element_type=jnp.float32)

    col_iota = lax.broadcasted_iota(jnp.int32, (nb, nb), 1)
    out_iota = lax.broadcasted_iota(jnp.int32, (nb, topk), 1)
    lut = jnp.zeros((nb, topk), jnp.int32)
    s = scores
    neg = jnp.float32(-jnp.inf)
    for t in range(topk):
        mx = jnp.max(s, axis=1, keepdims=True)
        cand = jnp.where(s == mx, col_iota, nb)
        idx = jnp.min(cand, axis=1)
        lut = jnp.where(out_iota == t, idx[:, None], lut)
        s = jnp.where(col_iota == idx[:, None], neg, s)
    lut_ref[0] = lut


def _attn_kernel(lut_ref, q_ref, k_ref, v_ref, o_ref, *, nb, topk, blk, d):
    scale = 1.0 / math.sqrt(d)
    npair = topk // 2

    def one_block(m):
        q_m = q_ref[0, pl.ds(m * blk, blk), :]
        s_parts = []
        v_parts = []
        for jp in range(npair):
            i0 = lut_ref[0, m, 2 * jp]
            i1 = lut_ref[0, m, 2 * jp + 1]
            kp = jnp.concatenate(
                [k_ref[0, pl.ds(i0 * blk, blk), :],
                 k_ref[0, pl.ds(i1 * blk, blk), :]], axis=0)
            vp = jnp.concatenate(
                [v_ref[0, pl.ds(i0 * blk, blk), :],
                 v_ref[0, pl.ds(i1 * blk, blk), :]], axis=0)
            s_parts.append(lax.dot_general(
                q_m, kp, (((1,), (1,)), ((), ())),
                preferred_element_type=jnp.float32))
            v_parts.append(vp)
        if topk % 2:
            i0 = lut_ref[0, m, topk - 1]
            s_parts.append(lax.dot_general(
                q_m, k_ref[0, pl.ds(i0 * blk, blk), :],
                (((1,), (1,)), ((), ())),
                preferred_element_type=jnp.float32))
            v_parts.append(v_ref[0, pl.ds(i0 * blk, blk), :])
        s_parts = [sp * scale for sp in s_parts]
        mx = s_parts[0].max(axis=1, keepdims=True)
        for sp in s_parts[1:]:
            mx = jnp.maximum(mx, sp.max(axis=1, keepdims=True))
        p_parts = [jnp.exp(sp - mx) for sp in s_parts]
        l = p_parts[0].sum(axis=1, keepdims=True)
        for pp in p_parts[1:]:
            l = l + pp.sum(axis=1, keepdims=True)
        linv = 1.0 / l
        o = jnp.zeros((blk, d), jnp.float32)
        for pp, vp in zip(p_parts, v_parts):
            o = o + lax.dot_general(
                pp * linv, vp, (((1,), (0,)), ((), ())),
                preferred_element_type=jnp.float32)
        o_ref[0, pl.ds(m * blk, blk), :] = o

    unroll = 2
    def body(i, _):
        for u in range(unroll):
            one_block(i * unroll + u)
        return 0

    lax.fori_loop(0, nb // unroll, body, 0)
    for r in range(nb % unroll):
        one_block(nb - 1 - r)


def kernel(q, k, v):
    B, H, L, D = q.shape
    nb = L // BLK
    topk = max(1, int(nb * TOPK_RATIO))
    BH = B * H
    qf = q.reshape(BH, L, D)
    kf = k.reshape(BH, L, D)
    vf = v.reshape(BH, L, D)

    head_spec = pl.BlockSpec((1, L, D), lambda i: (i, 0, 0))

    lut = pl.pallas_call(
        functools.partial(_lut_kernel, nb=nb, topk=topk, blk=BLK, d=D),
        grid=(BH,),
        in_specs=[head_spec, head_spec],
        out_specs=pl.BlockSpec((1, nb, topk), lambda i: (i, 0, 0)),
        out_shape=jax.ShapeDtypeStruct((BH, nb, topk), jnp.int32),
        compiler_params=pltpu.CompilerParams(
            dimension_semantics=("parallel",)),
    )(qf, kf)

    o = pl.pallas_call(
        functools.partial(_attn_kernel, nb=nb, topk=topk, blk=BLK, d=D),
        grid=(BH,),
        in_specs=[
            pl.BlockSpec((1, nb, topk), lambda i: (i, 0, 0),
                         memory_space=pltpu.SMEM),
            head_spec, head_spec, head_spec,
        ],
        out_specs=head_spec,
        out_shape=jax.ShapeDtypeStruct((BH, L, D), jnp.float32),
        compiler_params=pltpu.CompilerParams(
            dimension_semantics=("parallel",)),
    )(lut, qf, kf, vf)

    return o.reshape(B, H, L, D)


# speedup vs baseline: 1.2881x; 1.0941x over previous
import functools
import math

import jax
import jax.numpy as jnp
from jax import lax
from jax.experimental import pallas as pl
from jax.experimental.pallas import tpu as pltpu

BLK = 64
TOPK_RATIO = 0.125


def _lut_kernel(q_ref, k_ref, lut_ref, *, nb, topk, blk, d):
    qh = q_ref[0]
    kh = k_ref[0]
    pooled_q = qh.reshape(nb, blk, d).mean(axis=1)
    kmean = kh.mean(axis=0, keepdims=True)
    pooled_k = kh.reshape(nb, blk, d).mean(axis=1) - kmean
    scores = lax.dot_general(
        pooled_q, pooled_k, (((1,), (1,)), ((), ())),
        preferred_element_type=jnp.float32)

    col_iota = lax.broadcasted_iota(jnp.int32, (nb, nb), 1)
    out_iota = lax.broadcasted_iota(jnp.int32, (nb, topk), 1)
    lut = jnp.zeros((nb, topk), jnp.int32)
    s = scores
    neg = jnp.float32(-jnp.inf)
    for t in range(topk):
        mx = jnp.max(s, axis=1, keepdims=True)
        cand = jnp.where(s == mx, col_iota, nb)
        idx = jnp.min(cand, axis=1)
        lut = jnp.where(out_iota == t, idx[:, None], lut)
        s = jnp.where(col_iota == idx[:, None], neg, s)
    lut_ref[0] = lut


def _attn_kernel(lut_ref, q_ref, k_ref, v_ref, o_ref, *, nb, topk, blk, d):
    scale = 1.0 / math.sqrt(d)
    npair = topk // 2

    def one_block(m):
        q_m = q_ref[0, pl.ds(m * blk, blk), :]
        s_parts = []
        v_parts = []
        for jp in range(npair):
            i0 = lut_ref[0, m, 2 * jp]
            i1 = lut_ref[0, m, 2 * jp + 1]
            kp = jnp.concatenate(
                [k_ref[0, pl.ds(i0 * blk, blk), :],
                 k_ref[0, pl.ds(i1 * blk, blk), :]], axis=0)
            vp = jnp.concatenate(
                [v_ref[0, pl.ds(i0 * blk, blk), :],
                 v_ref[0, pl.ds(i1 * blk, blk), :]], axis=0)
            s_parts.append(lax.dot_general(
                q_m, kp, (((1,), (1,)), ((), ())),
                preferred_element_type=jnp.float32))
            v_parts.append(vp)
        if topk % 2:
            i0 = lut_ref[0, m, topk - 1]
            s_parts.append(lax.dot_general(
                q_m, k_ref[0, pl.ds(i0 * blk, blk), :],
                (((1,), (1,)), ((), ())),
                preferred_element_type=jnp.float32))
            v_parts.append(v_ref[0, pl.ds(i0 * blk, blk), :])
        s_parts = [sp * scale for sp in s_parts]
        mx = s_parts[0].max(axis=1, keepdims=True)
        for sp in s_parts[1:]:
            mx = jnp.maximum(mx, sp.max(axis=1, keepdims=True))
        p_parts = [jnp.exp(sp - mx) for sp in s_parts]
        l = p_parts[0].sum(axis=1, keepdims=True)
        for pp in p_parts[1:]:
            l = l + pp.sum(axis=1, keepdims=True)
        linv = 1.0 / l
        o = jnp.zeros((blk, d), jnp.float32)
        for pp, vp in zip(p_parts, v_parts):
            o = o + lax.dot_general(
                pp * linv, vp, (((1,), (0,)), ((), ())),
                preferred_element_type=jnp.float32)
        o_ref[0, pl.ds(m * blk, blk), :] = o

    unroll = 4
    def body(i, _):
        for u in range(unroll):
            one_block(i * unroll + u)
        return 0

    lax.fori_loop(0, nb // unroll, body, 0)
    for r in range(nb % unroll):
        one_block(nb - 1 - r)


def kernel(q, k, v):
    B, H, L, D = q.shape
    nb = L // BLK
    topk = max(1, int(nb * TOPK_RATIO))
    BH = B * H
    qf = q.reshape(BH, L, D)
    kf = k.reshape(BH, L, D)
    vf = v.reshape(BH, L, D)

    head_spec = pl.BlockSpec((1, L, D), lambda i: (i, 0, 0))

    lut = pl.pallas_call(
        functools.partial(_lut_kernel, nb=nb, topk=topk, blk=BLK, d=D),
        grid=(BH,),
        in_specs=[head_spec, head_spec],
        out_specs=pl.BlockSpec((1, nb, topk), lambda i: (i, 0, 0)),
        out_shape=jax.ShapeDtypeStruct((BH, nb, topk), jnp.int32),
        compiler_params=pltpu.CompilerParams(
            dimension_semantics=("parallel",)),
    )(qf, kf)

    o = pl.pallas_call(
        functools.partial(_attn_kernel, nb=nb, topk=topk, blk=BLK, d=D),
        grid=(BH,),
        in_specs=[
            pl.BlockSpec((1, nb, topk), lambda i: (i, 0, 0),
                         memory_space=pltpu.SMEM),
            head_spec, head_spec, head_spec,
        ],
        out_specs=head_spec,
        out_shape=jax.ShapeDtypeStruct((BH, L, D), jnp.float32),
        compiler_params=pltpu.CompilerParams(
            dimension_semantics=("parallel",)),
    )(lut, qf, kf, vf)

    return o.reshape(B, H, L, D)


# speedup vs baseline: 1.5326x; 1.1898x over previous
import functools
import math

import jax
import jax.numpy as jnp
from jax import lax
from jax.experimental import pallas as pl
from jax.experimental.pallas import tpu as pltpu

BLK = 64
TOPK_RATIO = 0.125


def _lut_kernel(q_ref, k_ref, lut_ref, *, nb, topk, blk, d):
    qh = q_ref[0]
    kh = k_ref[0]
    pooled_q = qh.reshape(nb, blk, d).mean(axis=1)
    kmean = kh.mean(axis=0, keepdims=True)
    pooled_k = kh.reshape(nb, blk, d).mean(axis=1) - kmean
    scores = lax.dot_general(
        pooled_q, pooled_k, (((1,), (1,)), ((), ())),
        preferred_element_type=jnp.float32)

    col_iota = lax.broadcasted_iota(jnp.int32, (nb, nb), 1)
    out_iota = lax.broadcasted_iota(jnp.int32, (nb, topk), 1)
    lut = jnp.zeros((nb, topk), jnp.int32)
    s = scores
    neg = jnp.float32(-jnp.inf)
    for t in range(topk):
        mx = jnp.max(s, axis=1, keepdims=True)
        cand = jnp.where(s == mx, col_iota, nb)
        idx = jnp.min(cand, axis=1)
        lut = jnp.where(out_iota == t, idx[:, None], lut)
        s = jnp.where(col_iota == idx[:, None], neg, s)
    lut_ref[0] = lut


def _attn_kernel(lut_ref, q_ref, k_ref, v_ref, o_ref, *, nb, topk, blk, d):
    scale = 1.0 / math.sqrt(d)
    npair = topk // 2

    def one_block(m):
        q_m = q_ref[0, pl.ds(m * blk, blk), :]
        k_cat = jnp.concatenate(
            [k_ref[0, pl.ds(lut_ref[0, m, j] * blk, blk), :]
             for j in range(topk)], axis=0)
        v_cat = jnp.concatenate(
            [v_ref[0, pl.ds(lut_ref[0, m, j] * blk, blk), :]
             for j in range(topk)], axis=0)
        s = lax.dot_general(
            q_m, k_cat, (((1,), (1,)), ((), ())),
            preferred_element_type=jnp.float32) * scale
        mx = s.max(axis=1, keepdims=True)
        p = jnp.exp(s - mx)
        linv = 1.0 / p.sum(axis=1, keepdims=True)
        o = lax.dot_general(
            p * linv, v_cat, (((1,), (0,)), ((), ())),
            preferred_element_type=jnp.float32)
        o_ref[0, pl.ds(m * blk, blk), :] = o

    unroll = 8
    def body(i, _):
        for u in range(unroll):
            one_block(i * unroll + u)
        return 0

    lax.fori_loop(0, nb // unroll, body, 0)
    for r in range(nb % unroll):
        one_block(nb - 1 - r)


def kernel(q, k, v):
    B, H, L, D = q.shape
    nb = L // BLK
    topk = max(1, int(nb * TOPK_RATIO))
    BH = B * H
    qf = q.reshape(BH, L, D)
    kf = k.reshape(BH, L, D)
    vf = v.reshape(BH, L, D)

    head_spec = pl.BlockSpec((1, L, D), lambda i: (i, 0, 0))

    lut = pl.pallas_call(
        functools.partial(_lut_kernel, nb=nb, topk=topk, blk=BLK, d=D),
        grid=(BH,),
        in_specs=[head_spec, head_spec],
        out_specs=pl.BlockSpec((1, nb, topk), lambda i: (i, 0, 0)),
        out_shape=jax.ShapeDtypeStruct((BH, nb, topk), jnp.int32),
        compiler_params=pltpu.CompilerParams(
            dimension_semantics=("parallel",)),
    )(qf, kf)

    o = pl.pallas_call(
        functools.partial(_attn_kernel, nb=nb, topk=topk, blk=BLK, d=D),
        grid=(BH,),
        in_specs=[
            pl.BlockSpec((1, nb, topk), lambda i: (i, 0, 0),
                         memory_space=pltpu.SMEM),
            head_spec, head_spec, head_spec,
        ],
        out_specs=head_spec,
        out_shape=jax.ShapeDtypeStruct((BH, L, D), jnp.float32),
        compiler_params=pltpu.CompilerParams(
            dimension_semantics=("parallel",)),
    )(lut, qf, kf, vf)

    return o.reshape(B, H, L, D)


# speedup vs baseline: 3.8980x; 2.5434x over previous
import functools
import math

import jax
import jax.numpy as jnp
from jax import lax
from jax.experimental import pallas as pl
from jax.experimental.pallas import tpu as pltpu

BLK = 64
TOPK_RATIO = 0.125


def _lut_kernel(q_ref, k_ref, lut_ref, *, nb, topk, blk, d):
    qh = q_ref[0]
    kh = k_ref[0]
    pooled_q = qh.reshape(nb, blk, d).mean(axis=1)
    kmean = kh.mean(axis=0, keepdims=True)
    pooled_k = kh.reshape(nb, blk, d).mean(axis=1) - kmean
    scores = lax.dot_general(
        pooled_q, pooled_k, (((1,), (1,)), ((), ())),
        preferred_element_type=jnp.float32)

    col_iota = lax.broadcasted_iota(jnp.int32, (nb, nb), 1)
    out_iota = lax.broadcasted_iota(jnp.int32, (nb, topk), 1)
    lut = jnp.zeros((nb, topk), jnp.int32)
    s = scores
    neg = jnp.float32(-jnp.inf)
    for t in range(topk):
        mx = jnp.max(s, axis=1, keepdims=True)
        cand = jnp.where(s == mx, col_iota, nb)
        idx = jnp.min(cand, axis=1)
        lut = jnp.where(out_iota == t, idx[:, None], lut)
        s = jnp.where(col_iota == idx[:, None], neg, s)
    lut_ref[0] = lut


def _attn_kernel(lut_ref, q_ref, k_ref, v_ref, o_ref, *, nb, topk, blk, d):
    scale = 1.0 / math.sqrt(d)
    npair = topk // 2

    def group_blocks(ms):
        staged = []
        for m in ms:
            q_m = q_ref[0, pl.ds(m * blk, blk), :]
            k_cat = jnp.concatenate(
                [k_ref[0, pl.ds(lut_ref[0, m, j] * blk, blk), :]
                 for j in range(topk)], axis=0)
            v_cat = jnp.concatenate(
                [v_ref[0, pl.ds(lut_ref[0, m, j] * blk, blk), :]
                 for j in range(topk)], axis=0)
            s = lax.dot_general(
                q_m, k_cat, (((1,), (1,)), ((), ())),
                preferred_element_type=jnp.float32) * scale
            staged.append((m, s, v_cat))
        for m, s, v_cat in staged:
            mx = s.max(axis=1, keepdims=True)
            p = jnp.exp(s - mx)
            linv = 1.0 / p.sum(axis=1, keepdims=True)
            o = lax.dot_general(
                (p * linv).astype(jnp.bfloat16), v_cat,
                (((1,), (0,)), ((), ())),
                preferred_element_type=jnp.float32)
            o_ref[0, pl.ds(m * blk, blk), :] = o

    group = 64
    def body(i, _):
        group_blocks([i * group + u for u in range(group)])
        return 0

    lax.fori_loop(0, nb // group, body, 0)
    if nb % group:
        group_blocks([nb - 1 - r for r in range(nb % group)])


def kernel(q, k, v):
    B, H, L, D = q.shape
    nb = L // BLK
    topk = max(1, int(nb * TOPK_RATIO))
    BH = B * H
    qf = q.reshape(BH, L, D)
    kf = k.reshape(BH, L, D)
    vf = v.reshape(BH, L, D)

    head_spec = pl.BlockSpec((1, L, D), lambda i: (i, 0, 0))

    lut = pl.pallas_call(
        functools.partial(_lut_kernel, nb=nb, topk=topk, blk=BLK, d=D),
        grid=(BH,),
        in_specs=[head_spec, head_spec],
        out_specs=pl.BlockSpec((1, nb, topk), lambda i: (i, 0, 0)),
        out_shape=jax.ShapeDtypeStruct((BH, nb, topk), jnp.int32),
        compiler_params=pltpu.CompilerParams(
            dimension_semantics=("parallel",)),
    )(qf, kf)

    o = pl.pallas_call(
        functools.partial(_attn_kernel, nb=nb, topk=topk, blk=BLK, d=D),
        grid=(BH,),
        in_specs=[
            pl.BlockSpec((1, nb, topk), lambda i: (i, 0, 0),
                         memory_space=pltpu.SMEM),
            head_spec, head_spec, head_spec,
        ],
        out_specs=head_spec,
        out_shape=jax.ShapeDtypeStruct((BH, L, D), jnp.float32),
        compiler_params=pltpu.CompilerParams(
            dimension_semantics=("parallel",)),
    )(lut, qf.astype(jnp.bfloat16), kf.astype(jnp.bfloat16),
      vf.astype(jnp.bfloat16))

    return o.reshape(B, H, L, D)


# speedup vs baseline: 4.9762x; 1.2766x over previous
import functools
import math

import jax
import jax.numpy as jnp
from jax import lax
from jax.experimental import pallas as pl
from jax.experimental.pallas import tpu as pltpu

BLK = 64
TOPK_RATIO = 0.125


def _lut_kernel(q_ref, k_ref, v_ref, lut_ref, qb_ref, kb_ref, vb_ref,
                *, nb, topk, blk, d):
    qh = q_ref[0]
    kh = k_ref[0]
    qb_ref[0] = qh.astype(jnp.bfloat16)
    kb_ref[0] = kh.astype(jnp.bfloat16)
    vb_ref[0] = v_ref[0].astype(jnp.bfloat16)
    pooled_q = qh.reshape(nb, blk, d).mean(axis=1)
    kmean = kh.mean(axis=0, keepdims=True)
    pooled_k = kh.reshape(nb, blk, d).mean(axis=1) - kmean
    scores = lax.dot_general(
        pooled_q, pooled_k, (((1,), (1,)), ((), ())),
        preferred_element_type=jnp.float32)

    col_iota = lax.broadcasted_iota(jnp.int32, (nb, nb), 1)
    out_iota = lax.broadcasted_iota(jnp.int32, (nb, topk), 1)
    lut = jnp.zeros((nb, topk), jnp.int32)
    s = scores
    neg = jnp.float32(-jnp.inf)
    for t in range(topk):
        mx = jnp.max(s, axis=1, keepdims=True)
        cand = jnp.where(s == mx, col_iota, nb)
        idx = jnp.min(cand, axis=1)
        lut = jnp.where(out_iota == t, idx[:, None], lut)
        s = jnp.where(col_iota == idx[:, None], neg, s)
    lut_ref[0] = lut


def _attn_kernel(lut_ref, q_ref, k_ref, v_ref, o_ref, *, nb, topk, blk, d):
    scale = 1.0 / math.sqrt(d)
    npair = topk // 2

    def group_blocks(ms):
        staged = []
        for m in ms:
            q_m = q_ref[0, pl.ds(m * blk, blk), :]
            k_cat = jnp.concatenate(
                [k_ref[0, pl.ds(lut_ref[0, m, j] * blk, blk), :]
                 for j in range(topk)], axis=0)
            v_cat = jnp.concatenate(
                [v_ref[0, pl.ds(lut_ref[0, m, j] * blk, blk), :]
                 for j in range(topk)], axis=0)
            s = lax.dot_general(
                q_m, k_cat, (((1,), (1,)), ((), ())),
                preferred_element_type=jnp.float32) * scale
            staged.append((m, s, v_cat))
        for m, s, v_cat in staged:
            mx = s.max(axis=1, keepdims=True)
            p = jnp.exp(s - mx)
            linv = 1.0 / p.sum(axis=1, keepdims=True)
            o = lax.dot_general(
                (p * linv).astype(jnp.bfloat16), v_cat,
                (((1,), (0,)), ((), ())),
                preferred_element_type=jnp.float32)
            o_ref[0, pl.ds(m * blk, blk), :] = o

    group = 64
    def body(i, _):
        group_blocks([i * group + u for u in range(group)])
        return 0

    lax.fori_loop(0, nb // group, body, 0)
    if nb % group:
        group_blocks([nb - 1 - r for r in range(nb % group)])


def kernel(q, k, v):
    B, H, L, D = q.shape
    nb = L // BLK
    topk = max(1, int(nb * TOPK_RATIO))
    BH = B * H
    qf = q.reshape(BH, L, D)
    kf = k.reshape(BH, L, D)
    vf = v.reshape(BH, L, D)

    head_spec = pl.BlockSpec((1, L, D), lambda i: (i, 0, 0))

    lut, qb, kb, vb = pl.pallas_call(
        functools.partial(_lut_kernel, nb=nb, topk=topk, blk=BLK, d=D),
        grid=(BH,),
        in_specs=[head_spec, head_spec, head_spec],
        out_specs=[
            pl.BlockSpec((1, nb, topk), lambda i: (i, 0, 0)),
            head_spec, head_spec, head_spec,
        ],
        out_shape=[
            jax.ShapeDtypeStruct((BH, nb, topk), jnp.int32),
            jax.ShapeDtypeStruct((BH, L, D), jnp.bfloat16),
            jax.ShapeDtypeStruct((BH, L, D), jnp.bfloat16),
            jax.ShapeDtypeStruct((BH, L, D), jnp.bfloat16),
        ],
        compiler_params=pltpu.CompilerParams(
            dimension_semantics=("parallel",)),
    )(qf, kf, vf)

    o = pl.pallas_call(
        functools.partial(_attn_kernel, nb=nb, topk=topk, blk=BLK, d=D),
        grid=(BH,),
        in_specs=[
            pl.BlockSpec((1, nb, topk), lambda i: (i, 0, 0),
                         memory_space=pltpu.SMEM),
            head_spec, head_spec, head_spec,
        ],
        out_specs=head_spec,
        out_shape=jax.ShapeDtypeStruct((BH, L, D), jnp.float32),
        compiler_params=pltpu.CompilerParams(
            dimension_semantics=("parallel",)),
    )(lut, qb, kb, vb)

    return o.reshape(B, H, L, D)
